# baseline (speedup 1.0000x reference)
import functools

import jax
import jax.numpy as jnp
from jax import lax
from jax.experimental import pallas as pl
from jax.experimental.pallas import tpu as pltpu
from jax.experimental.pallas import tpu_sc as plsc

B, C, N = 2, 64, 131072
H, W = 512, 512
HW = H * W
BN = B * N
BHW = B * HW

NC, NS = 2, 16
NW = NC * NS
CHUNK = 1024
NCHUNK = BHW // CHUNK
ROUNDS = NCHUNK // NW
LIST_CAP = 16384
CLIST_CAP = 1024
GB = 128
SEG_WIN = 4096
NWIN = BN // SEG_WIN

_NEG_INF = float("-inf")


def _pro_body(fa_ref, fb_ref, ixa_ref, iya_ref, ixb_ref, iyb_ref,
              fp_ref, seg_ref):
    fa = fa_ref[0]
    fb = fb_ref[0]
    fp_ref[:, 0:C] = fa.T
    fp_ref[:, C:2 * C] = fb.T
    vxa = jnp.clip(ixa_ref[0, 0], 0, H - 1)
    vya = jnp.clip(iya_ref[0, 0], 0, W - 1)
    seg_ref[0, 0, 0] = vxa * W + vya
    vxb = jnp.clip(ixb_ref[0, 0], 0, H - 1)
    vyb = jnp.clip(iyb_ref[0, 0], 0, W - 1)
    seg_ref[1, 0, 0] = HW + vxb * W + vyb


def _prologue(pcds_feat, ix, iy):
    nb = 8192
    nblk = N // nb
    ix3 = ix.reshape(B * nblk, 1, nb)
    iy3 = iy.reshape(B * nblk, 1, nb)
    return pl.pallas_call(
        _pro_body,
        grid=(nblk,),
        in_specs=[
            pl.BlockSpec((1, C, nb), lambda i: (0, 0, i)),
            pl.BlockSpec((1, C, nb), lambda i: (1, 0, i)),
            pl.BlockSpec((1, 1, nb), lambda i: (i, 0, 0)),
            pl.BlockSpec((1, 1, nb), lambda i: (i, 0, 0)),
            pl.BlockSpec((1, 1, nb), lambda i: (nblk + i, 0, 0)),
            pl.BlockSpec((1, 1, nb), lambda i: (nblk + i, 0, 0)),
        ],
        out_specs=[
            pl.BlockSpec((nb, 2 * C), lambda i: (i, 0)),
            pl.BlockSpec((2, 1, 1, nb), lambda i: (0, i, 0, 0)),
        ],
        out_shape=[
            jax.ShapeDtypeStruct((N, 2 * C), jnp.float32),
            jax.ShapeDtypeStruct((2, nblk, 1, nb), jnp.int32),
        ],
    )(pcds_feat[..., 0], pcds_feat[..., 0], ix3, iy3, ix3, iy3)


def _sc_body(feats_hbm, seg_hbm, temp_hbm,
             seg_buf, plist, cpid, clv, rows, accum, sem):
    wid = lax.axis_index("s") * NC + lax.axis_index("c")
    lanes = lax.iota(jnp.int32, 16)

    def _pf(i, _):
        cpid[pl.ds(i * 16, 16)] = jnp.zeros((16,), jnp.int32)
        return 0
    lax.fori_loop(0, CLIST_CAP // 16, _pf, 0)

    def _win(w, off):
        pltpu.sync_copy(seg_hbm.at[pl.ds(w * SEG_WIN, SEG_WIN)], seg_buf)

        def _vreg(j, off):
            s = seg_buf[pl.ds(j * 16, 16)]
            own = ((s >> 10) & (NW - 1)) == wid
            r = (s >> 15) & (ROUNDS - 1)
            lv = s & (CHUNK - 1)
            pid = w * SEG_WIN + j * 16 + lanes
            packed = (pid.astype(jnp.uint32) << 14) | \
                     ((r << 10) | lv).astype(jnp.uint32)
            key = jnp.where(own, jnp.uint32(0), jnp.uint32(1))
            _, sv = plsc.sort_key_val(key, packed)
            o = jnp.minimum(off, LIST_CAP - 16)
            plist[pl.ds(o, 16)] = sv
            return off + jnp.sum(own.astype(jnp.int32))
        return lax.fori_loop(0, SEG_WIN // 16, _vreg, off)

    m_total = jnp.minimum(lax.fori_loop(0, NWIN, _win, jnp.int32(0)), LIST_CAP)
    n_mv = (m_total + 15) // 16

    def _round(r, _):
        chunk = r * NW + wid
        pair_base = chunk * (CHUNK // 2)

        def _init(i, _):
            for u in range(4):
                for cg in range(8):
                    accum[i * 4 + u, pl.ds(cg * 16, 16)] = jnp.full(
                        (16,), _NEG_INF, jnp.float32)
            return 0
        lax.fori_loop(0, CHUNK // 2 // 4, _init, 0)

        def _scan(j, k):
            p = plist[pl.ds(j * 16, 16)]
            valid = (j * 16 + lanes) < m_total
            r_of = ((p >> 10) & jnp.uint32(ROUNDS - 1)).astype(jnp.int32)
            sel = jnp.logical_and(r_of == r, valid)
            key = jnp.where(sel, jnp.uint32(0), jnp.uint32(1))
            _, sv = plsc.sort_key_val(key, p)
            o = jnp.minimum(k, CLIST_CAP - 16)
            cpid[pl.ds(o, 16)] = ((sv >> 14) &
                                  jnp.uint32(N - 1)).astype(jnp.int32)
            clv[pl.ds(o, 16)] = ((sv & jnp.uint32(CHUNK - 1)) |
                                 ((sv >> 31) << 11)).astype(jnp.int32)
            return k + jnp.sum(sel.astype(jnp.int32))
        k_total = jnp.minimum(lax.fori_loop(0, n_mv, _scan, jnp.int32(0)),
                              CLIST_CAP)
        clv[pl.ds(k_total, 16)] = jnp.full((16,), CHUNK, jnp.int32)

        def _batch(g, _):
            pltpu.async_copy(
                feats_hbm.at[cpid.at[pl.ds(g * GB, GB)]], rows, sem).wait()
            cnt = jnp.minimum(k_total - g * GB, GB)

            def _grp(j, _):
                lvv = clv[pl.ds(g * GB + j * 16, 16)]
                for u in range(16):
                    e = lvv[u]
                    lv = e & 2047
                    hoff = ((e >> 11) & 1) << 6
                    trash = lv > CHUNK - 1
                    row = jnp.where(trash, CHUNK // 2, lv & (CHUNK // 2 - 1))
                    coff = jnp.where(trash, 0, (lv >> 9) << 6)
                    for cg in range(4):
                        a = accum[row, pl.ds(coff + cg * 16, 16)]
                        f = rows[j * 16 + u, pl.ds(hoff + cg * 16, 16)]
                        accum[row, pl.ds(coff + cg * 16, 16)] = \
                            jnp.maximum(a, f)
                return 0
            return lax.fori_loop(0, (cnt + 15) // 16, _grp, 0)
        lax.fori_loop(0, (k_total + GB - 1) // GB, _batch, 0)

        pltpu.sync_copy(accum.at[pl.ds(0, CHUNK // 2), :],
                        temp_hbm.at[pl.ds(pair_base, CHUNK // 2), :])
        return 0
    lax.fori_loop(0, ROUNDS, _round, 0)


@functools.partial(
    pl.kernel,
    out_type=jax.ShapeDtypeStruct((BHW // 2, 2 * C), jnp.float32),
    mesh=plsc.VectorSubcoreMesh(core_axis_name="c", subcore_axis_name="s",
                                num_cores=NC, num_subcores=NS),
    scratch_types=[
        pltpu.VMEM((SEG_WIN,), jnp.int32),
        pltpu.VMEM((LIST_CAP,), jnp.uint32),
        pltpu.VMEM((CLIST_CAP,), jnp.int32),
        pltpu.VMEM((CLIST_CAP + 16,), jnp.int32),
        pltpu.VMEM((GB, 2 * C), jnp.float32),
        pltpu.VMEM((CHUNK // 2 + 1, 2 * C), jnp.float32),
        pltpu.SemaphoreType.DMA,
    ],
    compiler_params=pltpu.CompilerParams(needs_layout_passes=False),
)
def _sc_kernel(feats_hbm, seg_hbm, temp_hbm,
               seg_buf, plist, cpid, clv, rows, accum, sem):
    _sc_body(feats_hbm, seg_hbm, temp_hbm,
             seg_buf, plist, cpid, clv, rows, accum, sem)


def _epi_body(temp_ref, out_ref):
    hc = CHUNK // 2
    for q in range(4):
        t = temp_ref[pl.ds(q * hc, hc), :]
        lo = t[:, 0:C]
        hi = t[:, C:2 * C]
        out_ref[0, :, pl.ds(q * CHUNK, hc)] = \
            jnp.where(jnp.isfinite(lo), lo, 0.0).T
        out_ref[0, :, pl.ds(q * CHUNK + hc, hc)] = \
            jnp.where(jnp.isfinite(hi), hi, 0.0).T


def _epilogue(temp):
    nck = 4
    per_b = NCHUNK // B // nck
    out = pl.pallas_call(
        _epi_body,
        grid=(NCHUNK // nck,),
        in_specs=[pl.BlockSpec((nck * CHUNK // 2, 2 * C),
                               lambda i: (i, 0))],
        out_specs=pl.BlockSpec((1, C, nck * CHUNK),
                               lambda i: (i // per_b, 0, i % per_b)),
        out_shape=jax.ShapeDtypeStruct((B, C, HW), jnp.float32),
    )(temp)
    return out.reshape(B, C, H, W)


def kernel(pcds_feat, pcds_ind):
    ix = pcds_ind[:, :, 0, 0]
    iy = pcds_ind[:, :, 1, 0]
    feats_p, seg = _prologue(pcds_feat, ix, iy)
    temp = _sc_kernel(feats_p, seg.reshape(BN))
    return _epilogue(temp)
